# baseline (speedup 1.0000x reference)
import functools

import jax
import jax.numpy as jnp
from jax.experimental import pallas as pl
from jax.experimental.pallas import tpu as pltpu


def _unisage_body(x_ref, a_ref, w_ref, b_ref, wout_ref, bout_ref,
                  out_ref, x_state, xT_bf, a_bf, m_acc, dacc,
                  *, n_layers, k_blocks, valid_last):
    l = pl.program_id(0)
    k = pl.program_id(1)
    d = x_ref.shape[1]

    @pl.when((l == 0) & (k == 0))
    def _():
        x_state[...] = x_ref[...]
        dacc[...] = jnp.zeros(dacc.shape, dacc.dtype)

    @pl.when(k == 0)
    def _():
        x_state[...] = jax.lax.dot_general(
            x_state[...].astype(jnp.bfloat16), w_ref[0].astype(jnp.bfloat16),
            (((1,), (1,)), ((), ())),
            preferred_element_type=jnp.float32,
        ) + b_ref[0]
        xT_bf[...] = jnp.swapaxes(x_state[...].astype(jnp.bfloat16), 0, 1)

    def step(w):
        a_bf[:, :w] = a_ref[:, :w].astype(jnp.bfloat16)
        m01T = jax.lax.dot_general(
            xT_bf[...], a_bf[:, :w], (((1,), (0,)), ((), ())),
            preferred_element_type=jnp.float32)
        contrib = jax.lax.dot_general(
            a_bf[:, :w], m01T.astype(jnp.bfloat16), (((1,), (1,)), ((), ())),
            preferred_element_type=jnp.float32)

        @pl.when(k == 0)
        def _():
            m_acc[...] = contrib

        @pl.when(k > 0)
        def _():
            m_acc[...] += contrib

        @pl.when(l == 0)
        def _():
            sums = {}
            for c in range(0, w, d):
                e = min(c + d, w)
                p = a_bf[:, c:e].astype(jnp.float32)
                width = e - c
                sums[width] = p if width not in sums else sums[width] + p
            for width, val in sums.items():
                dacc[:, :width] += val

    full_bk = a_ref.shape[1]
    if valid_last == full_bk:
        step(full_bk)
    else:
        @pl.when(k < k_blocks - 1)
        def _():
            step(full_bk)

        @pl.when(k == k_blocks - 1)
        def _():
            step(valid_last)

    @pl.when(k == k_blocks - 1)
    def _():
        @pl.when(l == 0)
        def _():
            dv = jnp.sum(dacc[...], axis=1, keepdims=True)
            x_state[...] = x_state[...] + m_acc[...] / dv
            dacc[:, 0:1] = dv

        @pl.when(l > 0)
        def _():
            x_state[...] = x_state[...] + m_acc[...] / dacc[:, 0:1]

    @pl.when((l == n_layers - 1) & (k == k_blocks - 1))
    def _():
        pooled = jnp.max(x_state[...], axis=0, keepdims=True)
        logit = jnp.sum(pooled * wout_ref[...], axis=1, keepdims=True)
        out_ref[...] = jax.nn.sigmoid(logit + bout_ref[...])


@jax.jit
def kernel(x_1, incidence_1, W0, b0, W1, b1, W_out, b_out):
    n, d = x_1.shape
    n_layers = 2
    bk = min(256, n)
    k_blocks = -(-n // bk)
    valid_last = n - (k_blocks - 1) * bk

    ws = jnp.stack([W0, W1])
    bs = jnp.stack([b0, b1]).reshape(n_layers, 1, d)
    bout = b_out.reshape(1, 1)

    grid = (n_layers, k_blocks)
    out = pl.pallas_call(
        functools.partial(_unisage_body, n_layers=n_layers,
                          k_blocks=k_blocks, valid_last=valid_last),
        grid=grid,
        in_specs=[
            pl.BlockSpec((n, d), lambda l, k: (0, 0)),
            pl.BlockSpec((n, bk), lambda l, k: (0, k)),
            pl.BlockSpec((1, d, d), lambda l, k: (l, 0, 0)),
            pl.BlockSpec((1, 1, d), lambda l, k: (l, 0, 0)),
            pl.BlockSpec((1, d), lambda l, k: (0, 0)),
            pl.BlockSpec((1, 1), lambda l, k: (0, 0)),
        ],
        out_specs=pl.BlockSpec((1, 1), lambda l, k: (0, 0)),
        out_shape=jax.ShapeDtypeStruct((1, 1), jnp.float32),
        scratch_shapes=[
            pltpu.VMEM((n, d), jnp.float32),
            pltpu.VMEM((d, n), jnp.bfloat16),
            pltpu.VMEM((n, bk), jnp.bfloat16),
            pltpu.VMEM((n, d), jnp.float32),
            pltpu.VMEM((n, d), jnp.float32),
        ],
        compiler_params=pltpu.CompilerParams(
            dimension_semantics=("arbitrary", "arbitrary"),
            vmem_limit_bytes=60 * 1024 * 1024,
        ),
    )(x_1, incidence_1, ws, bs, W_out, bout)
    return out.reshape(1)


# speedup vs baseline: 1.0876x; 1.0876x over previous
import functools

import jax
import jax.numpy as jnp
from jax.experimental import pallas as pl
from jax.experimental.pallas import tpu as pltpu


def _unisage_body(x_ref, a_ref, w_ref, b_ref, wout_ref, bout_ref,
                  out_ref, x_state, xT_bf, a_bf, m_acc, dacc,
                  *, n_layers, k_blocks, valid_last):
    l = pl.program_id(0)
    k = pl.program_id(1)
    d = x_ref.shape[1]

    @pl.when((l == 0) & (k == 0))
    def _():
        x_state[...] = x_ref[...]
        dacc[...] = jnp.zeros(dacc.shape, dacc.dtype)

    @pl.when(k == 0)
    def _():
        x_state[...] = jax.lax.dot_general(
            x_state[...].astype(jnp.bfloat16), w_ref[0].astype(jnp.bfloat16),
            (((1,), (1,)), ((), ())),
            preferred_element_type=jnp.float32,
        ) + b_ref[0]
        xT_bf[...] = x_state[...].astype(jnp.bfloat16)

    def step(w):
        a_bf[:, :w] = a_ref[:, :w].astype(jnp.bfloat16)
        m01 = jax.lax.dot_general(
            a_bf[:, :w], xT_bf[...], (((0,), (0,)), ((), ())),
            preferred_element_type=jnp.float32)
        contrib = jax.lax.dot_general(
            a_bf[:, :w], m01.astype(jnp.bfloat16), (((1,), (0,)), ((), ())),
            preferred_element_type=jnp.float32)

        @pl.when(k == 0)
        def _():
            m_acc[...] = contrib

        @pl.when(k > 0)
        def _():
            m_acc[...] += contrib

        @pl.when(l == 0)
        def _():
            sums = {}
            for c in range(0, w, d):
                e = min(c + d, w)
                p = a_bf[:, c:e].astype(jnp.float32)
                width = e - c
                sums[width] = p if width not in sums else sums[width] + p
            for width, val in sums.items():
                dacc[:, :width] += val

    full_bk = a_ref.shape[1]
    if valid_last == full_bk:
        step(full_bk)
    else:
        @pl.when(k < k_blocks - 1)
        def _():
            step(full_bk)

        @pl.when(k == k_blocks - 1)
        def _():
            step(valid_last)

    @pl.when(k == k_blocks - 1)
    def _():
        @pl.when(l == 0)
        def _():
            dv = jnp.sum(dacc[...], axis=1, keepdims=True)
            x_state[...] = x_state[...] + m_acc[...] / dv
            dacc[:, 0:1] = dv

        @pl.when(l > 0)
        def _():
            x_state[...] = x_state[...] + m_acc[...] / dacc[:, 0:1]

    @pl.when((l == n_layers - 1) & (k == k_blocks - 1))
    def _():
        pooled = jnp.max(x_state[...], axis=0, keepdims=True)
        logit = jnp.sum(pooled * wout_ref[...], axis=1, keepdims=True)
        out_ref[...] = jax.nn.sigmoid(logit + bout_ref[...])


@jax.jit
def kernel(x_1, incidence_1, W0, b0, W1, b1, W_out, b_out):
    n, d = x_1.shape
    n_layers = 2
    bk = min(256, n)
    k_blocks = -(-n // bk)
    valid_last = n - (k_blocks - 1) * bk

    ws = jnp.stack([W0, W1])
    bs = jnp.stack([b0, b1]).reshape(n_layers, 1, d)
    bout = b_out.reshape(1, 1)

    grid = (n_layers, k_blocks)
    out = pl.pallas_call(
        functools.partial(_unisage_body, n_layers=n_layers,
                          k_blocks=k_blocks, valid_last=valid_last),
        grid=grid,
        in_specs=[
            pl.BlockSpec((n, d), lambda l, k: (0, 0)),
            pl.BlockSpec((n, bk), lambda l, k: (0, k)),
            pl.BlockSpec((1, d, d), lambda l, k: (l, 0, 0)),
            pl.BlockSpec((1, 1, d), lambda l, k: (l, 0, 0)),
            pl.BlockSpec((1, d), lambda l, k: (0, 0)),
            pl.BlockSpec((1, 1), lambda l, k: (0, 0)),
        ],
        out_specs=pl.BlockSpec((1, 1), lambda l, k: (0, 0)),
        out_shape=jax.ShapeDtypeStruct((1, 1), jnp.float32),
        scratch_shapes=[
            pltpu.VMEM((n, d), jnp.float32),
            pltpu.VMEM((n, d), jnp.bfloat16),
            pltpu.VMEM((n, bk), jnp.bfloat16),
            pltpu.VMEM((n, d), jnp.float32),
            pltpu.VMEM((n, d), jnp.float32),
        ],
        compiler_params=pltpu.CompilerParams(
            dimension_semantics=("arbitrary", "arbitrary"),
            vmem_limit_bytes=60 * 1024 * 1024,
        ),
    )(x_1, incidence_1, ws, bs, W_out, bout)
    return out.reshape(1)
